# Initial kernel scaffold; baseline (speedup 1.0000x reference)
import jax
import jax.numpy as jnp
from jax.experimental import pallas as pl


def kernel(inputs):
    raise NotImplementedError("write your pallas kernel here")


# speedup vs baseline: 2.3249x; 2.3249x over previous
import functools

import jax
import jax.numpy as jnp
from jax import lax
from jax.experimental import pallas as pl
from jax.experimental.pallas import tpu as pltpu
from jax.experimental.pallas import tpu_sc as plsc

MS = 512
TL = MS * (MS + 1) // 2
BATCH = 128
TOTAL = BATCH * TL
NW = 32
ROWS_PER_W = MS // NW
WINDOW = 8080
LANES = 16
NCHUNK = MS // LANES


def _build():
  mesh = plsc.VectorSubcoreMesh(core_axis_name="c", subcore_axis_name="s")

  @functools.partial(
      pl.kernel,
      mesh=mesh,
      out_type=jax.ShapeDtypeStruct((BATCH, MS, MS), jnp.float32),
      scratch_types=[
          pltpu.VMEM((WINDOW,), jnp.float32),
          pltpu.VMEM((ROWS_PER_W, MS), jnp.float32),
      ],
  )
  def tri_kernel(in_hbm, out_hbm, win_ref, blk_ref):
    info = plsc.get_sparse_core_info()
    nc = info.num_cores
    wid = lax.axis_index("s") * nc + lax.axis_index("c")
    r0 = wid * ROWS_PER_W
    start0 = (r0 * (1023 - r0)) // 2
    al0 = (start0 // 8) * 8
    iota = lax.iota(jnp.int32, LANES)

    def bstep(b, carry):
      win_start = jnp.minimum(b * TL + al0, TOTAL - WINDOW)
      pltpu.sync_copy(in_hbm.at[pl.ds(win_start, WINDOW)], win_ref)
      for i in range(ROWS_PER_W):
        r = r0 + i
        row_base = b * TL + (r * (1023 - r)) // 2 - win_start
        for j in range(NCHUNK):
          cvec = j * LANES + iota
          v = win_ref[pl.ds(row_base + j * LANES, LANES)]
          v = jnp.where(cvec >= r, v, 0.0)
          blk_ref[i, pl.ds(j * LANES, LANES)] = v
      pltpu.sync_copy(blk_ref, out_hbm.at[b, pl.ds(r0, ROWS_PER_W)])
      return carry

    lax.fori_loop(0, BATCH, bstep, 0)

  return tri_kernel


@jax.jit
def kernel(inputs):
  return _build()(inputs.reshape(-1))


# speedup vs baseline: 4.2157x; 1.8133x over previous
import functools

import jax
import jax.numpy as jnp
from jax import lax
from jax.experimental import pallas as pl
from jax.experimental.pallas import tpu as pltpu
from jax.experimental.pallas import tpu_sc as plsc

MS = 512
TL = MS * (MS + 1) // 2
BATCH = 128
TOTAL = BATCH * TL
NW = 32
ROWS_PER_W = MS // NW
WINDOW = 8080
LANES = 16
NCHUNK = MS // LANES


def _build():
  mesh = plsc.VectorSubcoreMesh(core_axis_name="c", subcore_axis_name="s")

  @functools.partial(
      pl.kernel,
      mesh=mesh,
      out_type=jax.ShapeDtypeStruct((BATCH, MS, MS), jnp.float32),
      scratch_types=[
          pltpu.VMEM((WINDOW,), jnp.float32),
          pltpu.VMEM((WINDOW,), jnp.float32),
          pltpu.VMEM((ROWS_PER_W, MS), jnp.float32),
          pltpu.VMEM((ROWS_PER_W, MS), jnp.float32),
          pltpu.SemaphoreType.DMA,
          pltpu.SemaphoreType.DMA,
          pltpu.SemaphoreType.DMA,
          pltpu.SemaphoreType.DMA,
      ],
  )
  def tri_kernel(in_hbm, out_hbm, win0, win1, blk0, blk1,
                 in_sem0, in_sem1, out_sem0, out_sem1):
    info = plsc.get_sparse_core_info()
    nc = info.num_cores
    wins = (win0, win1)
    blks = (blk0, blk1)
    in_sems = (in_sem0, in_sem1)
    out_sems = (out_sem0, out_sem1)
    wid = lax.axis_index("s") * nc + lax.axis_index("c")
    r0 = wid * ROWS_PER_W
    start0 = (r0 * (1023 - r0)) // 2
    al0 = (start0 // 8) * 8
    iota = lax.iota(jnp.int32, LANES)

    def win_start(b):
      return jnp.minimum(b * TL + al0, TOTAL - WINDOW)

    def win_slice(b):
      return in_hbm.at[pl.ds(win_start(b), WINDOW)]

    def out_slice(b):
      return out_hbm.at[b, pl.ds(r0, ROWS_PER_W)]

    for p in range(2):
      pltpu.async_copy(win_slice(p), wins[p], in_sems[p])

    def gstep(g, carry):
      for p in range(2):
        b = 2 * g + p
        pltpu.make_async_copy(win_slice(b), wins[p], in_sems[p]).wait()

        @pl.when(g > 0)
        def _wait_out():
          pltpu.make_async_copy(
              blks[p], out_slice(b - 2), out_sems[p]).wait()

        ws = win_start(b)

        def row_body(i, c2, p=p, b=b, ws=ws):
          r = r0 + i
          row_base = b * TL + (r * (1023 - r)) // 2 - ws
          for j in range(NCHUNK):
            cvec = j * LANES + iota
            v = wins[p][pl.ds(row_base + j * LANES, LANES)]
            v = jnp.where(cvec >= r, v, 0.0)
            blks[p][i, pl.ds(j * LANES, LANES)] = v
          return c2

        lax.fori_loop(0, ROWS_PER_W, row_body, 0)
        pltpu.async_copy(blks[p], out_slice(b), out_sems[p])

        @pl.when(b + 2 < BATCH)
        def _prefetch():
          pltpu.async_copy(win_slice(b + 2), wins[p], in_sems[p])

      return carry

    lax.fori_loop(0, BATCH // 2, gstep, 0)

    for p in range(2):
      b = BATCH - 2 + p
      pltpu.make_async_copy(blks[p], out_slice(b), out_sems[p]).wait()

  return tri_kernel


@jax.jit
def kernel(inputs):
  return _build()(inputs.reshape(-1))


# speedup vs baseline: 5.6516x; 1.3406x over previous
import functools

import jax
import jax.numpy as jnp
from jax import lax
from jax.experimental import pallas as pl
from jax.experimental.pallas import tpu as pltpu
from jax.experimental.pallas import tpu_sc as plsc

MS = 512
TL = MS * (MS + 1) // 2
BATCH = 128
TOTAL = BATCH * TL
NW = 32
ROWS_PER_W = MS // NW
WINDOW = 8080
LANES = 16
NCHUNK = MS // LANES


def _build():
  mesh = plsc.VectorSubcoreMesh(core_axis_name="c", subcore_axis_name="s")

  @functools.partial(
      pl.kernel,
      mesh=mesh,
      out_type=jax.ShapeDtypeStruct((BATCH, MS, MS), jnp.float32),
      scratch_types=[
          pltpu.VMEM((WINDOW,), jnp.float32),
          pltpu.VMEM((WINDOW,), jnp.float32),
          pltpu.VMEM((ROWS_PER_W, MS), jnp.float32),
          pltpu.VMEM((ROWS_PER_W, MS), jnp.float32),
          pltpu.SemaphoreType.DMA,
          pltpu.SemaphoreType.DMA,
          pltpu.SemaphoreType.DMA,
          pltpu.SemaphoreType.DMA,
      ],
  )
  def tri_kernel(in_hbm, out_hbm, win0, win1, blk0, blk1,
                 in_sem0, in_sem1, out_sem0, out_sem1):
    info = plsc.get_sparse_core_info()
    nc = info.num_cores
    wins = (win0, win1)
    blks = (blk0, blk1)
    in_sems = (in_sem0, in_sem1)
    out_sems = (out_sem0, out_sem1)
    wid = lax.axis_index("s") * nc + lax.axis_index("c")
    r0 = wid * ROWS_PER_W
    start0 = (r0 * (1023 - r0)) // 2
    al0 = (start0 // 8) * 8
    iota = lax.iota(jnp.int32, LANES)

    def win_start(b):
      return jnp.minimum(b * TL + al0, TOTAL - WINDOW)

    def win_slice(b):
      return in_hbm.at[pl.ds(win_start(b), WINDOW)]

    def out_slice(b):
      return out_hbm.at[b, pl.ds(r0, ROWS_PER_W)]

    for p in range(2):
      pltpu.async_copy(win_slice(p), wins[p], in_sems[p])

    def gstep(g, carry):
      for p in range(2):
        b = 2 * g + p
        pltpu.make_async_copy(win_slice(b), wins[p], in_sems[p]).wait()

        @pl.when(g > 0)
        def _wait_out():
          pltpu.make_async_copy(
              blks[p], out_slice(b - 2), out_sems[p]).wait()

        ws = win_start(b)

        @plsc.parallel_loop(0, ROWS_PER_W, unroll=2)
        def row_body(i, p=p, b=b, ws=ws):
          r = r0 + i
          row_base = b * TL + (r * (1023 - r)) // 2 - ws
          for j in range(NCHUNK):
            cvec = j * LANES + iota
            v = wins[p][pl.ds(row_base + j * LANES, LANES)]
            v = jnp.where(cvec >= r, v, 0.0)
            blks[p][i, pl.ds(j * LANES, LANES)] = v
        pltpu.async_copy(blks[p], out_slice(b), out_sems[p])

        @pl.when(b + 2 < BATCH)
        def _prefetch():
          pltpu.async_copy(win_slice(b + 2), wins[p], in_sems[p])

      return carry

    lax.fori_loop(0, BATCH // 2, gstep, 0)

    for p in range(2):
      b = BATCH - 2 + p
      pltpu.make_async_copy(blks[p], out_slice(b), out_sems[p]).wait()

  return tri_kernel


@jax.jit
def kernel(inputs):
  return _build()(inputs.reshape(-1))


# speedup vs baseline: 7.1048x; 1.2571x over previous
import functools

import jax
import jax.numpy as jnp
from jax import lax
from jax.experimental import pallas as pl
from jax.experimental.pallas import tpu as pltpu
from jax.experimental.pallas import tpu_sc as plsc

MS = 512
TL = MS * (MS + 1) // 2
BATCH = 128
TOTAL = BATCH * TL
NW = 32
GROUP = 8
LANES = 16
NCHUNK = MS // LANES
WA = 4080
WB = 2288


def _start(r):
  return (r * (1023 - r)) // 2


def _build():
  mesh = plsc.VectorSubcoreMesh(core_axis_name="c", subcore_axis_name="s")

  @functools.partial(
      pl.kernel,
      mesh=mesh,
      out_type=jax.ShapeDtypeStruct((BATCH, MS, MS), jnp.float32),
      scratch_types=[
          pltpu.VMEM((WA,), jnp.float32),
          pltpu.VMEM((WA,), jnp.float32),
          pltpu.VMEM((WB,), jnp.float32),
          pltpu.VMEM((WB,), jnp.float32),
          pltpu.VMEM((2 * GROUP, MS), jnp.float32),
          pltpu.VMEM((2 * GROUP, MS), jnp.float32),
          pltpu.SemaphoreType.DMA,
          pltpu.SemaphoreType.DMA,
          pltpu.SemaphoreType.DMA,
          pltpu.SemaphoreType.DMA,
      ],
  )
  def tri_kernel(in_hbm, out_hbm, winA0, winA1, winB0, winB1, blk0, blk1,
                 in_sem0, in_sem1, out_sem0, out_sem1):
    info = plsc.get_sparse_core_info()
    nc = info.num_cores
    winsA = (winA0, winA1)
    winsB = (winB0, winB1)
    blks = (blk0, blk1)
    in_sems = (in_sem0, in_sem1)
    out_sems = (out_sem0, out_sem1)
    wid = lax.axis_index("s") * nc + lax.axis_index("c")
    r0A = GROUP * wid
    r0B = MS - GROUP - GROUP * wid
    alA = (_start(r0A) // 8) * 8
    alB = (_start(r0B) // 8) * 8
    iota = lax.iota(jnp.int32, LANES)
    zero = jnp.zeros((LANES,), jnp.float32)

    def winA_slice(b):
      return in_hbm.at[pl.ds(b * TL + alA, WA)]

    def wsB(b):
      return jnp.minimum(b * TL + alB, TOTAL - WB)

    def winB_slice(b):
      return in_hbm.at[pl.ds(wsB(b), WB)]

    def outA_slice(b):
      return out_hbm.at[b, pl.ds(r0A, GROUP)]

    def outB_slice(b):
      return out_hbm.at[b, pl.ds(r0B, GROUP)]

    def zrow(t, carry):
      for p in range(2):
        for j in range(NCHUNK):
          blks[p][t, pl.ds(j * LANES, LANES)] = zero
      return carry

    lax.fori_loop(0, 2 * GROUP, zrow, 0)

    for p in range(2):
      pltpu.async_copy(winA_slice(p), winsA[p], in_sems[p])
      pltpu.async_copy(winB_slice(p), winsB[p], in_sems[p])

    def emit_row(blk, trow, r, win, base):
      jm = r // LANES
      rm = r - jm * LANES
      off = jm * LANES
      v = win[pl.ds(base + off, LANES)]
      v = jnp.where(iota >= rm, v, 0.0)
      blk[trow, pl.ds(pl.multiple_of(off, LANES), LANES)] = v

      @plsc.parallel_loop(jm + 1, NCHUNK, unroll=2)
      def _cp(j):
        o = pl.multiple_of(j * LANES, LANES)
        blk[trow, pl.ds(o, LANES)] = win[pl.ds(base + j * LANES, LANES)]

    def gstep(g, carry):
      for p in range(2):
        b = 2 * g + p
        pltpu.make_async_copy(winA_slice(b), winsA[p], in_sems[p]).wait()
        pltpu.make_async_copy(winB_slice(b), winsB[p], in_sems[p]).wait()

        @pl.when(g > 0)
        def _wait_out():
          pltpu.make_async_copy(blks[p].at[pl.ds(0, GROUP)],
                                outA_slice(b - 2), out_sems[p]).wait()
          pltpu.make_async_copy(blks[p].at[pl.ds(GROUP, GROUP)],
                                outB_slice(b - 2), out_sems[p]).wait()

        baseB0 = b * TL - wsB(b)
        for t in range(GROUP):
          rA = r0A + t
          emit_row(blks[p], t, rA, winsA[p], _start(rA) - alA)
          rB = r0B + t
          emit_row(blks[p], GROUP + t, rB, winsB[p], baseB0 + _start(rB))

        pltpu.async_copy(blks[p].at[pl.ds(0, GROUP)], outA_slice(b),
                         out_sems[p])
        pltpu.async_copy(blks[p].at[pl.ds(GROUP, GROUP)], outB_slice(b),
                         out_sems[p])

        @pl.when(b + 2 < BATCH)
        def _prefetch():
          pltpu.async_copy(winA_slice(b + 2), winsA[p], in_sems[p])
          pltpu.async_copy(winB_slice(b + 2), winsB[p], in_sems[p])

      return carry

    lax.fori_loop(0, BATCH // 2, gstep, 0)

    for p in range(2):
      b = BATCH - 2 + p
      pltpu.make_async_copy(blks[p].at[pl.ds(0, GROUP)], outA_slice(b),
                            out_sems[p]).wait()
      pltpu.make_async_copy(blks[p].at[pl.ds(GROUP, GROUP)], outB_slice(b),
                            out_sems[p]).wait()

  return tri_kernel


@jax.jit
def kernel(inputs):
  return _build()(inputs.reshape(-1))
